# Initial kernel scaffold; baseline (speedup 1.0000x reference)
import jax
import jax.numpy as jnp
from jax.experimental import pallas as pl


def kernel(x, rel_pos_table, W):
    raise NotImplementedError("write your pallas kernel here")


# speedup vs baseline: 7.2323x; 7.2323x over previous
import functools

import jax
import jax.numpy as jnp
from jax import lax
from jax.experimental import pallas as pl
from jax.experimental.pallas import tpu as pltpu
from jax.experimental.pallas import tpu_sc as plsc

KNN = 9
GRID_N = 100
N = GRID_N * GRID_N
NPAD = 10240
C = 128
RBLK = 128
NBLK = NPAD // RBLK
NW = 32
NODES_PW = NPAD // NW
SUB = 8
NCHUNK = NODES_PW // SUB


def _knn_body(xb_ref, xt_ref, ef_ref, nbr_ref, xf2_ref):
    i = pl.program_id(0)
    xb = xb_ref[...]
    xt = xt_ref[...]
    xf2_ref[...] = xb + ef_ref[...]
    g = jnp.dot(xb, xt, preferred_element_type=jnp.float32)
    sqc = jnp.sum(xt * xt, axis=0, keepdims=True)
    d = sqc - 2.0 * g
    col = lax.broadcasted_iota(jnp.int32, (RBLK, NPAD), 1)
    row = i * RBLK + lax.broadcasted_iota(jnp.int32, (RBLK, NPAD), 0)
    inf = jnp.float32(jnp.inf)
    d = jnp.where((col == row) | (col >= N), inf, d)
    big = jnp.int32(2**30)
    idxs = []
    for _ in range(KNN):
        val = jnp.min(d, axis=1, keepdims=True)
        idx = jnp.min(jnp.where(d <= val, col, big), axis=1, keepdims=True)
        idxs.append(idx)
        d = jnp.where(col == idx, inf, d)
    nbr = jnp.concatenate(idxs + [jnp.zeros((RBLK, 16 - KNN), jnp.int32)],
                          axis=1)
    nbr_ref[0] = nbr


def _knn_call(xpad, xt, efpad):
    return pl.pallas_call(
        _knn_body,
        grid=(NBLK,),
        in_specs=[
            pl.BlockSpec((RBLK, C), lambda i: (i, 0)),
            pl.BlockSpec((C, NPAD), lambda i: (0, 0)),
            pl.BlockSpec((RBLK, C), lambda i: (i, 0)),
        ],
        out_specs=[
            pl.BlockSpec((1, RBLK, 16), lambda i: (i, 0, 0)),
            pl.BlockSpec((RBLK, C), lambda i: (i, 0)),
        ],
        out_shape=[
            jax.ShapeDtypeStruct((NBLK, RBLK, 16), jnp.int32),
            jax.ShapeDtypeStruct((NPAD, C), jnp.float32),
        ],
    )(xpad, xt, efpad)


@functools.lru_cache(maxsize=1)
def _aggr_kernel():
    mesh = plsc.VectorSubcoreMesh(core_axis_name="c", subcore_axis_name="s")

    @functools.partial(
        pl.kernel,
        mesh=mesh,
        out_type=jax.ShapeDtypeStruct((NPAD, C), jnp.float32),
        scratch_types=[
            pltpu.VMEM((NODES_PW * KNN,), jnp.int32),
            pltpu.VMEM((SUB * KNN, C), jnp.float32),
            pltpu.VMEM((SUB, C), jnp.float32),
            pltpu.VMEM((SUB, C), jnp.float32),
            pltpu.SemaphoreType.DMA,
        ],
    )
    def aggr(xf2_hbm, idx_hbm, out_hbm, idx_v, rows_v, self_v, out_v, sem):
        wid = lax.axis_index("s") * 2 + lax.axis_index("c")
        base = wid * NODES_PW
        pltpu.sync_copy(idx_hbm.at[pl.ds(base * KNN, NODES_PW * KNN)], idx_v)

        def chunk(c, carry):
            nb = base + c * SUB
            pltpu.async_copy(
                xf2_hbm.at[idx_v.at[pl.ds(c * (SUB * KNN), SUB * KNN)]],
                rows_v, sem).wait()
            pltpu.sync_copy(xf2_hbm.at[pl.ds(nb, SUB)], self_v)
            for s in range(SUB):
                for v in range(C // 16):
                    sl = pl.ds(v * 16, 16)
                    a = self_v[s, sl]
                    acc = a
                    for j in range(KNN):
                        acc = jnp.maximum(acc, rows_v[s * KNN + j, sl])
                    out_v[s, sl] = acc - a
            pltpu.sync_copy(out_v, out_hbm.at[pl.ds(nb, SUB)])
            return carry

        lax.fori_loop(0, NCHUNK, chunk, 0)

    return aggr


def _mm_body(a_ref, w_ref, o_ref):
    o_ref[...] = lax.dot_general(
        w_ref[...], a_ref[...], (((1,), (1,)), ((), ())),
        preferred_element_type=jnp.float32)


def _mm_call(aggr, w):
    blk = 512
    return pl.pallas_call(
        _mm_body,
        grid=(NPAD // blk,),
        in_specs=[
            pl.BlockSpec((blk, C), lambda i: (i, 0)),
            pl.BlockSpec((C, C), lambda i: (0, 0)),
        ],
        out_specs=pl.BlockSpec((C, blk), lambda i: (0, i)),
        out_shape=jax.ShapeDtypeStruct((C, NPAD), jnp.float32),
    )(aggr, w)


def kernel(x, rel_pos_table, W):
    xf = x[0].T
    xpad = jnp.pad(xf, ((0, NPAD - N), (0, 0)))
    xt = jnp.pad(x[0], ((0, 0), (0, NPAD - N)))
    gidx = jnp.arange(GRID_N)
    rel = (gidx[:, None] - gidx[None, :] + (GRID_N - 1)).reshape(-1)
    ef = jnp.take(rel_pos_table, rel, axis=0)
    efpad = jnp.pad(ef, ((0, NPAD - N), (0, 0)))
    nbr3, xf2 = _knn_call(xpad, xt, efpad)
    flat_idx = nbr3.reshape(NPAD, 16)[:, :KNN].reshape(-1)
    aggr = _aggr_kernel()(xf2, flat_idx)
    out_t = _mm_call(aggr, W)
    return out_t[:, :N].reshape(1, C, N)


# speedup vs baseline: 9.0872x; 1.2565x over previous
import functools

import jax
import jax.numpy as jnp
from jax import lax
from jax.experimental import pallas as pl
from jax.experimental.pallas import tpu as pltpu
from jax.experimental.pallas import tpu_sc as plsc

KNN = 9
GRID_N = 100
N = GRID_N * GRID_N
NPAD = 10240
C = 128
RBLK = 256
NBLK = NPAD // RBLK
NW = 32
NODES_PW = NPAD // NW
SUB = 8
NCHUNK = NODES_PW // SUB


def _knn_body(xb_ref, xt_ref, ef_ref, nbr_ref, xf2_ref):
    i = pl.program_id(0)
    xb = xb_ref[...]
    xt = xt_ref[...]
    xf2_ref[...] = xb + ef_ref[...]
    g = jnp.dot(xb, xt, preferred_element_type=jnp.float32)
    sqc = jnp.sum(xt * xt, axis=0, keepdims=True)
    d = sqc - 2.0 * g
    col = lax.broadcasted_iota(jnp.int32, (RBLK, NPAD), 1)
    row = i * RBLK + lax.broadcasted_iota(jnp.int32, (RBLK, NPAD), 0)
    inf = jnp.float32(jnp.inf)
    d = jnp.where((col == row) | (col >= N), inf, d)
    colf = col.astype(jnp.float32)
    bigf = jnp.float32(2**30)
    idxs = []
    for _ in range(KNN):
        val = jnp.min(d, axis=1, keepdims=True)
        m = d <= val
        idx = jnp.min(jnp.where(m, colf, bigf), axis=1, keepdims=True)
        idxs.append(idx.astype(jnp.int32))
        d = jnp.where(m, inf, d)
    nbr = jnp.concatenate(idxs + [jnp.zeros((RBLK, 16 - KNN), jnp.int32)],
                          axis=1)
    nbr_ref[0] = nbr


def _knn_call(xpad, xt, efpad):
    return pl.pallas_call(
        _knn_body,
        grid=(NBLK,),
        in_specs=[
            pl.BlockSpec((RBLK, C), lambda i: (i, 0)),
            pl.BlockSpec((C, NPAD), lambda i: (0, 0)),
            pl.BlockSpec((RBLK, C), lambda i: (i, 0)),
        ],
        out_specs=[
            pl.BlockSpec((1, RBLK, 16), lambda i: (i, 0, 0)),
            pl.BlockSpec((RBLK, C), lambda i: (i, 0)),
        ],
        out_shape=[
            jax.ShapeDtypeStruct((NBLK, RBLK, 16), jnp.int32),
            jax.ShapeDtypeStruct((NPAD, C), jnp.float32),
        ],
    )(xpad, xt, efpad)


@functools.lru_cache(maxsize=1)
def _aggr_kernel():
    mesh = plsc.VectorSubcoreMesh(core_axis_name="c", subcore_axis_name="s")

    @functools.partial(
        pl.kernel,
        mesh=mesh,
        out_type=jax.ShapeDtypeStruct((NPAD, C), jnp.float32),
        scratch_types=[
            pltpu.VMEM((NODES_PW * KNN,), jnp.int32),
            pltpu.VMEM((SUB * KNN, C), jnp.float32),
            pltpu.VMEM((SUB, C), jnp.float32),
            pltpu.VMEM((SUB, C), jnp.float32),
            pltpu.SemaphoreType.DMA,
        ],
    )
    def aggr(xf2_hbm, idx_hbm, out_hbm, idx_v, rows_v, self_v, out_v, sem):
        wid = lax.axis_index("s") * 2 + lax.axis_index("c")
        base = wid * NODES_PW
        pltpu.sync_copy(idx_hbm.at[pl.ds(base * KNN, NODES_PW * KNN)], idx_v)

        def chunk(c, carry):
            nb = base + c * SUB
            pltpu.async_copy(
                xf2_hbm.at[idx_v.at[pl.ds(c * (SUB * KNN), SUB * KNN)]],
                rows_v, sem).wait()
            pltpu.sync_copy(xf2_hbm.at[pl.ds(nb, SUB)], self_v)
            for s in range(SUB):
                for v in range(C // 16):
                    sl = pl.ds(v * 16, 16)
                    a = self_v[s, sl]
                    acc = a
                    for j in range(KNN):
                        acc = jnp.maximum(acc, rows_v[s * KNN + j, sl])
                    out_v[s, sl] = acc - a
            pltpu.sync_copy(out_v, out_hbm.at[pl.ds(nb, SUB)])
            return carry

        lax.fori_loop(0, NCHUNK, chunk, 0)

    return aggr


def _mm_body(a_ref, w_ref, o_ref):
    o_ref[...] = lax.dot_general(
        w_ref[...], a_ref[...], (((1,), (1,)), ((), ())),
        preferred_element_type=jnp.float32)


def _mm_call(aggr, w):
    blk = 512
    return pl.pallas_call(
        _mm_body,
        grid=(NPAD // blk,),
        in_specs=[
            pl.BlockSpec((blk, C), lambda i: (i, 0)),
            pl.BlockSpec((C, C), lambda i: (0, 0)),
        ],
        out_specs=pl.BlockSpec((C, blk), lambda i: (0, i)),
        out_shape=jax.ShapeDtypeStruct((C, NPAD), jnp.float32),
    )(aggr, w)


def kernel(x, rel_pos_table, W):
    xf = x[0].T
    xpad = jnp.pad(xf, ((0, NPAD - N), (0, 0)))
    xt = jnp.pad(x[0], ((0, 0), (0, NPAD - N)))
    gidx = jnp.arange(GRID_N)
    rel = (gidx[:, None] - gidx[None, :] + (GRID_N - 1)).reshape(-1)
    ef = jnp.take(rel_pos_table, rel, axis=0)
    efpad = jnp.pad(ef, ((0, NPAD - N), (0, 0)))
    nbr3, xf2 = _knn_call(xpad, xt, efpad)
    flat_idx = nbr3.reshape(NPAD, 16)[:, :KNN].reshape(-1)
    aggr = _aggr_kernel()(xf2, flat_idx)
    out_t = _mm_call(aggr, W)
    return out_t[:, :N].reshape(1, C, N)


# speedup vs baseline: 9.1791x; 1.0101x over previous
import functools

import jax
import jax.numpy as jnp
from jax import lax
from jax.experimental import pallas as pl
from jax.experimental.pallas import tpu as pltpu
from jax.experimental.pallas import tpu_sc as plsc

KNN = 9
GRID_N = 100
N = GRID_N * GRID_N
NPAD = 10240
C = 128
RBLK = 256
NBLK = NPAD // RBLK
NW = 32
NODES_PW = NPAD // NW
SUB = 8
NCHUNK = NODES_PW // SUB


def _knn_body(xb_ref, xt_ref, ef_ref, nbr_ref, xf2_ref):
    xb = xb_ref[...]
    xt = xt_ref[...]
    xf2_ref[...] = xb + ef_ref[...]
    g = jnp.dot(xb, xt, preferred_element_type=jnp.float32)
    i = pl.program_id(0)
    sqc = jnp.sum(xt * xt, axis=0, keepdims=True)
    d = sqc - 2.0 * g
    col = lax.broadcasted_iota(jnp.int32, (RBLK, NPAD), 1)
    row = i * RBLK + lax.broadcasted_iota(jnp.int32, (RBLK, NPAD), 0)
    inf = jnp.float32(jnp.inf)
    d = jnp.where((col == row) | (col >= N), inf, d)
    colf = col.astype(jnp.float32)
    bigf = jnp.float32(2**30)
    idxs = []
    for _ in range(KNN):
        val = jnp.min(d, axis=1, keepdims=True)
        m = d <= val
        idx = jnp.min(jnp.where(m, colf, bigf), axis=1, keepdims=True)
        idxs.append(idx.astype(jnp.int32))
        d = jnp.where(m, inf, d)
    nbr = jnp.concatenate(idxs + [jnp.zeros((RBLK, 16 - KNN), jnp.int32)],
                          axis=1)
    nbr_ref[0] = nbr


def _knn_call(xpad, xt, efpad):
    return pl.pallas_call(
        _knn_body,
        grid=(NBLK,),
        in_specs=[
            pl.BlockSpec((RBLK, C), lambda i: (i, 0)),
            pl.BlockSpec((C, NPAD), lambda i: (0, 0)),
            pl.BlockSpec((RBLK, C), lambda i: (i, 0)),
        ],
        out_specs=[
            pl.BlockSpec((1, RBLK, 16), lambda i: (i, 0, 0)),
            pl.BlockSpec((RBLK, C), lambda i: (i, 0)),
        ],
        out_shape=[
            jax.ShapeDtypeStruct((NBLK, RBLK, 16), jnp.int32),
            jax.ShapeDtypeStruct((NPAD, C), jnp.float32),
        ],
    )(xpad, xt, efpad)


@functools.lru_cache(maxsize=1)
def _aggr_kernel():
    mesh = plsc.VectorSubcoreMesh(core_axis_name="c", subcore_axis_name="s")

    @functools.partial(
        pl.kernel,
        mesh=mesh,
        out_type=jax.ShapeDtypeStruct((NPAD, C), jnp.float32),
        scratch_types=[
            pltpu.VMEM((NODES_PW * KNN,), jnp.int32),
            pltpu.VMEM((2, SUB * KNN, C), jnp.float32),
            pltpu.VMEM((2, SUB, C), jnp.float32),
            pltpu.VMEM((2, SUB, C), jnp.float32),
            pltpu.SemaphoreType.DMA,
            pltpu.SemaphoreType.DMA,
            pltpu.SemaphoreType.DMA,
            pltpu.SemaphoreType.DMA,
        ],
    )
    def aggr(xf2_hbm, idx_hbm, out_hbm, idx_v, rows_v, self_v, out_v,
             sem_g0, sem_g1, sem_w0, sem_w1):
        wid = lax.axis_index("s") * 2 + lax.axis_index("c")
        base = wid * NODES_PW
        pltpu.sync_copy(idx_hbm.at[pl.ds(base * KNN, NODES_PW * KNN)], idx_v)
        sem_g = (sem_g0, sem_g1)
        sem_w = (sem_w0, sem_w1)

        def gather(c, b):
            pltpu.async_copy(
                xf2_hbm.at[idx_v.at[pl.ds(c * (SUB * KNN), SUB * KNN)]],
                rows_v.at[b], sem_g[b])
            pltpu.async_copy(
                xf2_hbm.at[pl.ds(base + c * SUB, SUB)], self_v.at[b], sem_g[b])

        gather(0, 0)
        gather(1, 1)

        def do_chunk(c, b, p):
            pltpu.make_async_copy(xf2_hbm.at[pl.ds(0, SUB * KNN)],
                                  rows_v.at[b], sem_g[b]).wait()
            pltpu.make_async_copy(xf2_hbm.at[pl.ds(0, SUB)],
                                  self_v.at[b], sem_g[b]).wait()
            @pl.when(p > 0)
            def _():
                pltpu.make_async_copy(xf2_hbm.at[pl.ds(0, SUB)],
                                      out_v.at[b], sem_w[b]).wait()
            for s in range(SUB):
                for v in range(C // 16):
                    sl = pl.ds(v * 16, 16)
                    a = self_v[b, s, sl]
                    acc = a
                    for j in range(KNN):
                        acc = jnp.maximum(acc, rows_v[b, s * KNN + j, sl])
                    out_v[b, s, sl] = acc - a
            pltpu.async_copy(out_v.at[b], out_hbm.at[pl.ds(base + c * SUB, SUB)],
                             sem_w[b])
            @pl.when(c + 2 < NCHUNK)
            def _():
                gather(c + 2, b)

        def pair(p, carry):
            do_chunk(2 * p, 0, p)
            do_chunk(2 * p + 1, 1, p)
            return carry

        lax.fori_loop(0, NCHUNK // 2, pair, 0)
        pltpu.make_async_copy(xf2_hbm.at[pl.ds(0, SUB)], out_v.at[0],
                              sem_w[0]).wait()
        pltpu.make_async_copy(xf2_hbm.at[pl.ds(0, SUB)], out_v.at[1],
                              sem_w[1]).wait()

    return aggr


def _mm_body(a_ref, w_ref, o_ref):
    o_ref[...] = lax.dot_general(
        w_ref[...], a_ref[...], (((1,), (1,)), ((), ())),
        preferred_element_type=jnp.float32)


def _mm_call(aggr, w):
    blk = 512
    return pl.pallas_call(
        _mm_body,
        grid=(NPAD // blk,),
        in_specs=[
            pl.BlockSpec((blk, C), lambda i: (i, 0)),
            pl.BlockSpec((C, C), lambda i: (0, 0)),
        ],
        out_specs=pl.BlockSpec((C, blk), lambda i: (0, i)),
        out_shape=jax.ShapeDtypeStruct((C, NPAD), jnp.float32),
    )(aggr, w)


def kernel(x, rel_pos_table, W):
    xt = jnp.pad(x[0], ((0, 0), (0, NPAD - N)))
    xpad = jnp.pad(x[0].T, ((0, NPAD - N), (0, 0)))
    gidx = jnp.arange(GRID_N)
    rel = (gidx[:, None] - gidx[None, :] + (GRID_N - 1)).reshape(-1)
    ef = jnp.take(rel_pos_table, rel, axis=0)
    efpad = jnp.pad(ef, ((0, NPAD - N), (0, 0)))
    nbr3, xf2 = _knn_call(xpad, xt, efpad)
    flat_idx = nbr3.reshape(NPAD, 16)[:, :KNN].reshape(-1)
    aggr = _aggr_kernel()(xf2, flat_idx)
    out_t = _mm_call(aggr, W)
    return out_t[:, :N].reshape(1, C, N)
